# baseline (speedup 1.0000x reference)
import numpy as np

import jax
import jax.numpy as jnp
from jax.experimental import pallas as pl
from jax.experimental.pallas import tpu as pltpu

_F = 56 * 56
_FP = _F + 64
_FC_TK = 8192


def _stage1_maps():
    order = []
    for e in (0, 1):
        for f in (0, 1):
            for a in ((0,) if e == 0 else (0, 1)):
                for b in ((0,) if f == 0 else (0, 1)):
                    order.append((e, f, a, b))
    idx = np.zeros((len(order), 32, 48), np.int32)
    msk = np.zeros((len(order), 32, 48), np.float32)
    for g, (e, f, a, b) in enumerate(order):
        for o in range(4):
            di, dj = o // 2, o % 2
            for p in range(4):
                khp = 4 * a + p - 2 * e
                kh = khp - di
                if not (0 <= khp < 4 and 0 <= kh < 3):
                    continue
                for q in range(4):
                    kwp = 4 * b + q - 2 * f
                    kw = kwp - dj
                    if not (0 <= kwp < 4 and 0 <= kw < 3):
                        continue
                    for co in range(6):
                        for ci in range(3):
                            r, c = 8 * o + co, 12 * p + 3 * q + ci
                            idx[g, r, c] = co * 27 + (kh * 3 + kw) * 3 + ci
                            msk[g, r, c] = 1.0
    return order, idx, msk


def _stage2_maps():
    idx = np.zeros((4, 64, 32), np.int32)
    msk = np.zeros((4, 64, 32), np.float32)
    for a in (0, 1):
        for b in (0, 1):
            g = 2 * a + b
            for o in range(4):
                di, dj = o // 2, o % 2
                for e in (0, 1):
                    kh = 2 * a + e - di
                    if not 0 <= kh < 3:
                        continue
                    for f in (0, 1):
                        kw = 2 * b + f - dj
                        if not 0 <= kw < 3:
                            continue
                        for co in range(16):
                            for ci in range(6):
                                r, c = 16 * o + co, 8 * (2 * e + f) + ci
                                idx[g, r, c] = co * 54 + (kh * 3 + kw) * 6 + ci
                                msk[g, r, c] = 1.0
    return idx, msk


_S1_ORDER, _S1_IDX, _S1_MSK = _stage1_maps()
_S2_IDX, _S2_MSK = _stage2_maps()


def _conv_pool_kernel(x_ref, w1s_ref, b1_ref, w2s_ref, b2_ref, o_ref, y_ref):
    y_ref[...] = jnp.zeros_like(y_ref)

    g = 0
    for e in (0, 1):
        for f in (0, 1):
            acc = None
            for a in ((0,) if e == 0 else (0, 1)):
                for b in ((0,) if f == 0 else (0, 1)):
                    s0 = 56 * a + b
                    t = jnp.dot(w1s_ref[g], x_ref[0, :, s0:s0 + _F],
                                preferred_element_type=jnp.float32)
                    acc = t if acc is None else acc + t
                    g += 1
            m = jnp.maximum(jnp.maximum(acc[0:6], acc[8:14]),
                            jnp.maximum(acc[16:22], acc[24:30]))
            y_ref[8 * (2 * e + f):8 * (2 * e + f) + 6, :_F] = (
                jnp.maximum(m + b1_ref[...], 0.0))

    acc2 = None
    for a in (0, 1):
        for b in (0, 1):
            s0 = 56 * a + b
            t = jnp.dot(w2s_ref[2 * a + b], y_ref[:, s0:s0 + _F],
                        preferred_element_type=jnp.float32)
            acc2 = t if acc2 is None else acc2 + t
    m2 = jnp.maximum(jnp.maximum(acc2[0:16], acc2[16:32]),
                     jnp.maximum(acc2[32:48], acc2[48:64]))
    o_ref[0] = jnp.maximum(m2 + b2_ref[...], 0.0).astype(o_ref.dtype)


def _fc_head_kernel(x_ref, w1_ref, b1_ref, w2_ref, b2_ref, w3_ref, b3_ref,
                    w4_ref, b4_ref, o_ref, acc_ref):
    k = pl.program_id(0)
    part = jnp.dot(x_ref[...], w1_ref[...], preferred_element_type=jnp.float32)

    @pl.when(k == 0)
    def _():
        acc_ref[...] = part

    @pl.when(k > 0)
    def _():
        acc_ref[...] += part

    @pl.when(k == pl.num_programs(0) - 1)
    def _():
        h = jnp.maximum(acc_ref[...] + b1_ref[...], 0.0)
        h = jnp.maximum(jnp.dot(h, w2_ref[...],
                                preferred_element_type=jnp.float32)
                        + b2_ref[...], 0.0)
        h = jnp.maximum(jnp.dot(h, w3_ref[...],
                                preferred_element_type=jnp.float32)
                        + b3_ref[...], 0.0)
        z = jnp.dot(h, w4_ref[...],
                    preferred_element_type=jnp.float32) + b4_ref[...]
        zm = jnp.max(z, axis=-1, keepdims=True)
        o_ref[...] = ((z - zm) - jnp.log(
            jnp.sum(jnp.exp(z - zm), axis=-1, keepdims=True))).astype(o_ref.dtype)


def kernel(x_nchw, w1t, b1, w2t, b2, wf1t, bf1, wf2t, bf2, wf3t, bf3,
           wf4t, bf4):
    x = x_nchw.astype(jnp.float32)
    n = x.shape[0]

    xt = x.reshape(n, 3, 56, 4, 56, 4).transpose(0, 3, 5, 1, 2, 4)
    xt = xt.reshape(n, 48, _F)
    xt = jnp.pad(xt, ((0, 0), (0, 0), (0, _FP - _F)))

    w1p = jnp.pad(w1t.reshape(6, 3, 3, 3), ((0, 0), (3, 3), (3, 3), (0, 0)))
    g1 = []
    for e, f, a, b in _S1_ORDER:
        rows = []
        for o in range(4):
            di, dj = o // 2, o % 2
            sh = di + 2 * e - 4 * a
            sw = dj + 2 * f - 4 * b
            blk = w1p[:, 3 - sh:7 - sh, 3 - sw:7 - sw, :].reshape(6, 48)
            rows.append(jnp.pad(blk, ((0, 2), (0, 0))))
        g1.append(jnp.concatenate(rows, axis=0))
    w1s = jnp.stack(g1)

    w2p = jnp.pad(w2t.reshape(16, 3, 3, 6), ((0, 0), (1, 2), (1, 2), (0, 0)))
    g2 = []
    for a in (0, 1):
        for b in (0, 1):
            rows = []
            for o in range(4):
                di, dj = o // 2, o % 2
                sh = di - 2 * a
                sw = dj - 2 * b
                blk = w2p[:, 1 - sh:3 - sh, 1 - sw:3 - sw, :]
                blk = jnp.pad(blk, ((0, 0), (0, 0), (0, 0), (0, 2)))
                rows.append(blk.reshape(16, 32))
            g2.append(jnp.concatenate(rows, axis=0))
    w2s = jnp.stack(g2)

    z = pl.pallas_call(
        _conv_pool_kernel,
        out_shape=jax.ShapeDtypeStruct((n, 16, _F), jnp.bfloat16),
        grid_spec=pltpu.PrefetchScalarGridSpec(
            num_scalar_prefetch=0,
            grid=(n,),
            in_specs=[
                pl.BlockSpec((1, 48, _FP), lambda i: (i, 0, 0)),
                pl.BlockSpec((9, 32, 48), lambda i: (0, 0, 0)),
                pl.BlockSpec((6, 1), lambda i: (0, 0)),
                pl.BlockSpec((4, 64, 32), lambda i: (0, 0, 0)),
                pl.BlockSpec((16, 1), lambda i: (0, 0)),
            ],
            out_specs=pl.BlockSpec((1, 16, _F), lambda i: (i, 0, 0)),
            scratch_shapes=[pltpu.VMEM((32, _FP), jnp.float32)],
        ),
        compiler_params=pltpu.CompilerParams(
            dimension_semantics=("parallel",),
            vmem_limit_bytes=32 * 1024 * 1024,
        ),
    )(xt, w1s, b1.reshape(6, 1), w2s, b2.reshape(16, 1))

    feats = z.reshape(n, 16, 56, 56)[:, :, :54, :54].reshape(n, 16 * 54 * 54)
    kp = wf1t.shape[0]
    feats = jnp.pad(feats, ((0, 0), (0, kp - feats.shape[1])))

    return pl.pallas_call(
        _fc_head_kernel,
        out_shape=jax.ShapeDtypeStruct((n, wf4t.shape[1]), jnp.float32),
        grid_spec=pltpu.PrefetchScalarGridSpec(
            num_scalar_prefetch=0,
            grid=(kp // _FC_TK,),
            in_specs=[
                pl.BlockSpec((n, _FC_TK), lambda k: (0, k)),
                pl.BlockSpec((_FC_TK, wf1t.shape[1]), lambda k: (k, 0)),
                pl.BlockSpec((1, wf1t.shape[1]), lambda k: (0, 0)),
                pl.BlockSpec(wf2t.shape, lambda k: (0, 0)),
                pl.BlockSpec((1, wf2t.shape[1]), lambda k: (0, 0)),
                pl.BlockSpec(wf3t.shape, lambda k: (0, 0)),
                pl.BlockSpec((1, wf3t.shape[1]), lambda k: (0, 0)),
                pl.BlockSpec(wf4t.shape, lambda k: (0, 0)),
                pl.BlockSpec((1, wf4t.shape[1]), lambda k: (0, 0)),
            ],
            out_specs=pl.BlockSpec((n, wf4t.shape[1]), lambda k: (0, 0)),
            scratch_shapes=[pltpu.VMEM((n, wf1t.shape[1]), jnp.float32)],
        ),
        compiler_params=pltpu.CompilerParams(
            dimension_semantics=("arbitrary",),
            vmem_limit_bytes=32 * 1024 * 1024,
        ),
    )(feats, wf1t.astype(jnp.bfloat16), bf1.reshape(1, -1),
      wf2t, bf2.reshape(1, -1),
      wf3t, bf3.reshape(1, -1), wf4t, bf4.reshape(1, -1))


# speedup vs baseline: 1.0146x; 1.0146x over previous
import numpy as np

import jax
import jax.numpy as jnp
from jax.experimental import pallas as pl
from jax.experimental.pallas import tpu as pltpu

_F = 56 * 56
_FP = _F + 64
_FC_TK = 8192


def _stage1_maps():
    order = []
    for e in (0, 1):
        for f in (0, 1):
            for a in ((0,) if e == 0 else (0, 1)):
                for b in ((0,) if f == 0 else (0, 1)):
                    order.append((e, f, a, b))
    idx = np.zeros((len(order), 32, 48), np.int32)
    msk = np.zeros((len(order), 32, 48), np.float32)
    for g, (e, f, a, b) in enumerate(order):
        for o in range(4):
            di, dj = o // 2, o % 2
            for p in range(4):
                khp = 4 * a + p - 2 * e
                kh = khp - di
                if not (0 <= khp < 4 and 0 <= kh < 3):
                    continue
                for q in range(4):
                    kwp = 4 * b + q - 2 * f
                    kw = kwp - dj
                    if not (0 <= kwp < 4 and 0 <= kw < 3):
                        continue
                    for co in range(6):
                        for ci in range(3):
                            r, c = 8 * o + co, 12 * p + 3 * q + ci
                            idx[g, r, c] = co * 27 + (kh * 3 + kw) * 3 + ci
                            msk[g, r, c] = 1.0
    return order, idx, msk


def _stage2_maps():
    idx = np.zeros((4, 64, 32), np.int32)
    msk = np.zeros((4, 64, 32), np.float32)
    for a in (0, 1):
        for b in (0, 1):
            g = 2 * a + b
            for o in range(4):
                di, dj = o // 2, o % 2
                for e in (0, 1):
                    kh = 2 * a + e - di
                    if not 0 <= kh < 3:
                        continue
                    for f in (0, 1):
                        kw = 2 * b + f - dj
                        if not 0 <= kw < 3:
                            continue
                        for co in range(16):
                            for ci in range(6):
                                r, c = 16 * o + co, 8 * (2 * e + f) + ci
                                idx[g, r, c] = co * 54 + (kh * 3 + kw) * 6 + ci
                                msk[g, r, c] = 1.0
    return idx, msk


_S1_ORDER, _S1_IDX, _S1_MSK = _stage1_maps()
_S2_IDX, _S2_MSK = _stage2_maps()


def _conv_pool_kernel(x_ref, w1s_ref, b1_ref, w2s_ref, b2_ref, o_ref, y_ref):
    y_ref[...] = jnp.zeros_like(y_ref)
    xv = jnp.pad(x_ref[0], ((0, 0), (0, _FP - _F)))

    g = 0
    for e in (0, 1):
        for f in (0, 1):
            acc = None
            for a in ((0,) if e == 0 else (0, 1)):
                for b in ((0,) if f == 0 else (0, 1)):
                    s0 = 56 * a + b
                    t = jnp.dot(w1s_ref[g], xv[:, s0:s0 + _F],
                                preferred_element_type=jnp.float32)
                    acc = t if acc is None else acc + t
                    g += 1
            m = jnp.maximum(jnp.maximum(acc[0:6], acc[8:14]),
                            jnp.maximum(acc[16:22], acc[24:30]))
            y_ref[8 * (2 * e + f):8 * (2 * e + f) + 6, :_F] = (
                jnp.maximum(m + b1_ref[...], 0.0))

    acc2 = None
    for a in (0, 1):
        for b in (0, 1):
            s0 = 56 * a + b
            t = jnp.dot(w2s_ref[2 * a + b], y_ref[:, s0:s0 + _F],
                        preferred_element_type=jnp.float32)
            acc2 = t if acc2 is None else acc2 + t
    m2 = jnp.maximum(jnp.maximum(acc2[0:16], acc2[16:32]),
                     jnp.maximum(acc2[32:48], acc2[48:64]))
    o_ref[0] = jnp.maximum(m2 + b2_ref[...], 0.0).astype(o_ref.dtype)


def _fc_head_kernel(x_ref, w1_ref, b1_ref, w2_ref, b2_ref, w3_ref, b3_ref,
                    w4_ref, b4_ref, o_ref, acc_ref):
    k = pl.program_id(0)
    part = jnp.dot(x_ref[...], w1_ref[...], preferred_element_type=jnp.float32)

    @pl.when(k == 0)
    def _():
        acc_ref[...] = part

    @pl.when(k > 0)
    def _():
        acc_ref[...] += part

    @pl.when(k == pl.num_programs(0) - 1)
    def _():
        h = jnp.maximum(acc_ref[...] + b1_ref[...], 0.0)
        h = jnp.maximum(jnp.dot(h, w2_ref[...],
                                preferred_element_type=jnp.float32)
                        + b2_ref[...], 0.0)
        h = jnp.maximum(jnp.dot(h, w3_ref[...],
                                preferred_element_type=jnp.float32)
                        + b3_ref[...], 0.0)
        z = jnp.dot(h, w4_ref[...],
                    preferred_element_type=jnp.float32) + b4_ref[...]
        zm = jnp.max(z, axis=-1, keepdims=True)
        o_ref[...] = ((z - zm) - jnp.log(
            jnp.sum(jnp.exp(z - zm), axis=-1, keepdims=True))).astype(o_ref.dtype)


def kernel(x_nchw, w1t, b1, w2t, b2, wf1t, bf1, wf2t, bf2, wf3t, bf3,
           wf4t, bf4):
    x = x_nchw.astype(jnp.float32)
    n = x.shape[0]

    xt = x.reshape(n, 3, 56, 4, 56, 4).transpose(0, 3, 5, 1, 2, 4)
    xt = xt.reshape(n, 48, _F)

    w1p = jnp.pad(w1t.reshape(6, 3, 3, 3), ((0, 0), (3, 3), (3, 3), (0, 0)))
    g1 = []
    for e, f, a, b in _S1_ORDER:
        rows = []
        for o in range(4):
            di, dj = o // 2, o % 2
            sh = di + 2 * e - 4 * a
            sw = dj + 2 * f - 4 * b
            blk = w1p[:, 3 - sh:7 - sh, 3 - sw:7 - sw, :].reshape(6, 48)
            rows.append(jnp.pad(blk, ((0, 2), (0, 0))))
        g1.append(jnp.concatenate(rows, axis=0))
    w1s = jnp.stack(g1)

    w2p = jnp.pad(w2t.reshape(16, 3, 3, 6), ((0, 0), (1, 2), (1, 2), (0, 0)))
    g2 = []
    for a in (0, 1):
        for b in (0, 1):
            rows = []
            for o in range(4):
                di, dj = o // 2, o % 2
                sh = di - 2 * a
                sw = dj - 2 * b
                blk = w2p[:, 1 - sh:3 - sh, 1 - sw:3 - sw, :]
                blk = jnp.pad(blk, ((0, 0), (0, 0), (0, 0), (0, 2)))
                rows.append(blk.reshape(16, 32))
            g2.append(jnp.concatenate(rows, axis=0))
    w2s = jnp.stack(g2)

    z = pl.pallas_call(
        _conv_pool_kernel,
        out_shape=jax.ShapeDtypeStruct((n, 16, _F), jnp.bfloat16),
        grid_spec=pltpu.PrefetchScalarGridSpec(
            num_scalar_prefetch=0,
            grid=(n,),
            in_specs=[
                pl.BlockSpec((1, 48, _F), lambda i: (i, 0, 0)),
                pl.BlockSpec((9, 32, 48), lambda i: (0, 0, 0)),
                pl.BlockSpec((6, 1), lambda i: (0, 0)),
                pl.BlockSpec((4, 64, 32), lambda i: (0, 0, 0)),
                pl.BlockSpec((16, 1), lambda i: (0, 0)),
            ],
            out_specs=pl.BlockSpec((1, 16, _F), lambda i: (i, 0, 0)),
            scratch_shapes=[pltpu.VMEM((32, _FP), jnp.float32)],
        ),
        compiler_params=pltpu.CompilerParams(
            dimension_semantics=("parallel",),
            vmem_limit_bytes=32 * 1024 * 1024,
        ),
    )(xt, w1s, b1.reshape(6, 1), w2s, b2.reshape(16, 1))

    feats = z.reshape(n, 16, 56, 56)[:, :, :54, :54].reshape(n, 16 * 54 * 54)
    kp = wf1t.shape[0]
    feats = jnp.pad(feats, ((0, 0), (0, kp - feats.shape[1])))

    return pl.pallas_call(
        _fc_head_kernel,
        out_shape=jax.ShapeDtypeStruct((n, wf4t.shape[1]), jnp.float32),
        grid_spec=pltpu.PrefetchScalarGridSpec(
            num_scalar_prefetch=0,
            grid=(kp // _FC_TK,),
            in_specs=[
                pl.BlockSpec((n, _FC_TK), lambda k: (0, k)),
                pl.BlockSpec((_FC_TK, wf1t.shape[1]), lambda k: (k, 0)),
                pl.BlockSpec((1, wf1t.shape[1]), lambda k: (0, 0)),
                pl.BlockSpec(wf2t.shape, lambda k: (0, 0)),
                pl.BlockSpec((1, wf2t.shape[1]), lambda k: (0, 0)),
                pl.BlockSpec(wf3t.shape, lambda k: (0, 0)),
                pl.BlockSpec((1, wf3t.shape[1]), lambda k: (0, 0)),
                pl.BlockSpec(wf4t.shape, lambda k: (0, 0)),
                pl.BlockSpec((1, wf4t.shape[1]), lambda k: (0, 0)),
            ],
            out_specs=pl.BlockSpec((n, wf4t.shape[1]), lambda k: (0, 0)),
            scratch_shapes=[pltpu.VMEM((n, wf1t.shape[1]), jnp.float32)],
        ),
        compiler_params=pltpu.CompilerParams(
            dimension_semantics=("arbitrary",),
            vmem_limit_bytes=32 * 1024 * 1024,
        ),
    )(feats, wf1t.astype(jnp.bfloat16), bf1.reshape(1, -1),
      wf2t, bf2.reshape(1, -1),
      wf3t, bf3.reshape(1, -1), wf4t, bf4.reshape(1, -1))
